# Initial kernel scaffold; baseline (speedup 1.0000x reference)
import jax
import jax.numpy as jnp
from jax.experimental import pallas as pl


def kernel(node_features, edge_index, adj_values, W, b):
    raise NotImplementedError("write your pallas kernel here")


# speedup vs baseline: 3.5374x; 3.5374x over previous
import functools

import jax
import jax.numpy as jnp
from jax import lax
from jax.experimental import pallas as pl
from jax.experimental.pallas import tpu as pltpu
from jax.experimental.pallas import tpu_sc as plsc

N_NODES = 10000
N_EDGES = 320000
F = 128

NC = 2
NS = 16
NW = NC * NS
CHUNK = 128
NCHUNK = 80
E_PAD = NW * NCHUNK * CHUNK
ROWS_PER_TILE = N_NODES // NS
LANES = 16
FV = F // LANES


def _spmm_body(x_hbm, src_hbm, dst_hbm, a_hbm, out_hbm,
               src_v, dst_v, a_v, rows_v, acc_sh, sem):
    cid = lax.axis_index("c")
    sid = lax.axis_index("s")
    wid = cid * NS + sid

    pltpu.sync_copy(src_hbm.at[wid], src_v)
    pltpu.sync_copy(dst_hbm.at[wid], dst_v)
    pltpu.sync_copy(a_hbm.at[wid], a_v)

    zero = jnp.zeros((LANES,), jnp.float32)

    def zrow(i, carry):
        for k in range(FV):
            rows_v[i, pl.ds(k * LANES, LANES)] = zero
        return carry

    lax.fori_loop(0, CHUNK, zrow, 0)
    for t in range(ROWS_PER_TILE // CHUNK):
        pltpu.sync_copy(rows_v, acc_sh.at[pl.ds(sid * ROWS_PER_TILE + t * CHUNK, CHUNK)])
    rem = ROWS_PER_TILE % CHUNK
    if rem:
        pltpu.sync_copy(
            rows_v.at[pl.ds(0, rem)],
            acc_sh.at[pl.ds(sid * ROWS_PER_TILE + (ROWS_PER_TILE // CHUNK) * CHUNK, rem)])
    plsc.subcore_barrier()

    def chunk_body(j, carry):
        pltpu.async_copy(x_hbm.at[src_v.at[j]], rows_v, sem).wait()

        def scale(g, c):
            av16 = a_v[j, pl.ds(g * LANES, LANES)]
            for i in range(LANES):
                avb = jnp.full((LANES,), av16[i], dtype=jnp.float32)
                e = g * LANES + i
                for k in range(FV):
                    sl = (e, pl.ds(k * LANES, LANES))
                    rows_v[sl] = rows_v[sl] * avb
            return c

        lax.fori_loop(0, CHUNK // LANES, scale, 0)

        pltpu.sync_copy(rows_v, acc_sh.at[dst_v.at[j]], add=True)
        return carry

    lax.fori_loop(0, NCHUNK, chunk_body, 0)
    plsc.subcore_barrier()

    W_ROWS = 624
    pltpu.sync_copy(acc_sh.at[pl.ds(sid * W_ROWS, W_ROWS)],
                    out_hbm.at[cid, pl.ds(sid * W_ROWS, W_ROWS)])

    @pl.when(sid == NS - 1)
    def _tail():
        pltpu.sync_copy(acc_sh.at[pl.ds(NS * W_ROWS, N_NODES - NS * W_ROWS)],
                        out_hbm.at[cid, pl.ds(NS * W_ROWS, N_NODES - NS * W_ROWS)])


@jax.jit
def _spmm_sc(x, src3, dst3, a3):
    mesh = plsc.VectorSubcoreMesh(core_axis_name="c", subcore_axis_name="s")
    return pl.kernel(
        _spmm_body,
        out_type=jax.ShapeDtypeStruct((NC, N_NODES, F), jnp.float32),
        mesh=mesh,
        scratch_types=[
            pltpu.VMEM((NCHUNK, CHUNK), jnp.int32),
            pltpu.VMEM((NCHUNK, CHUNK), jnp.int32),
            pltpu.VMEM((NCHUNK, CHUNK), jnp.float32),
            pltpu.VMEM((CHUNK, F), jnp.float32),
            pltpu.VMEM_SHARED((N_NODES, F), jnp.float32),
            pltpu.SemaphoreType.DMA,
        ],
    )(x, src3, dst3, a3)


def _finish_body(p_ref, w_ref, b_ref, o_ref):
    acc = p_ref[0] + p_ref[1]
    y = jnp.dot(acc, w_ref[...], preferred_element_type=jnp.float32)
    y = y + b_ref[...]
    o_ref[...] = jnp.where(y >= 0, y, 0.01 * y)


@jax.jit
def _finish_tc(partials, W, b):
    return pl.pallas_call(
        _finish_body,
        out_shape=jax.ShapeDtypeStruct((N_NODES, F), jnp.float32),
    )(partials, W, b.reshape(1, F))


def kernel(node_features, edge_index, adj_values, W, b):
    dst = edge_index[0].astype(jnp.int32)
    src = edge_index[1].astype(jnp.int32)
    pad = E_PAD - N_EDGES
    dst3 = jnp.pad(dst, (0, pad)).reshape(NW, NCHUNK, CHUNK)
    src3 = jnp.pad(src, (0, pad)).reshape(NW, NCHUNK, CHUNK)
    a3 = jnp.pad(adj_values, (0, pad)).reshape(NW, NCHUNK, CHUNK)
    partials = _spmm_sc(node_features, src3, dst3, a3)
    return _finish_tc(partials, W, b)


# speedup vs baseline: 4.2751x; 1.2085x over previous
import jax
import jax.numpy as jnp
from jax import lax
from jax.experimental import pallas as pl
from jax.experimental.pallas import tpu as pltpu
from jax.experimental.pallas import tpu_sc as plsc

N_NODES = 10000
N_EDGES = 320000
F = 128

NC = 2
NS = 16
NW = NC * NS
CHUNK = 64
NCHUNK = 160
HALF = NCHUNK // 2
E_PAD = NW * NCHUNK * CHUNK
LANES = 16
FV = F // LANES
NBUF = 2


def _spmm_body(x_hbm, src_hbm, dst_hbm, a_hbm, out_hbm,
               src_v, dst_v, a_v, rows0, rows1,
               acc_sh, gsems):
    cid = lax.axis_index("c")
    sid = lax.axis_index("s")
    wid = cid * NS + sid
    rows = (rows0, rows1)

    def stage(h):
        pltpu.sync_copy(src_hbm.at[wid, pl.ds(h * HALF, HALF)], src_v)
        pltpu.sync_copy(dst_hbm.at[wid, pl.ds(h * HALF, HALF)], dst_v)
        pltpu.sync_copy(a_hbm.at[wid, pl.ds(h * HALF, HALF)], a_v)

    stage(0)

    zero = jnp.zeros((LANES,), jnp.float32)

    def zrow(i, carry):
        for k in range(FV):
            rows0[i, pl.ds(k * LANES, LANES)] = zero
        return carry

    lax.fori_loop(0, CHUNK, zrow, 0)
    RPT = N_NODES // NS
    for t in range(RPT // CHUNK):
        pltpu.sync_copy(rows0, acc_sh.at[pl.ds(sid * RPT + t * CHUNK, CHUNK)])
    rem = RPT % CHUNK
    if rem:
        pltpu.sync_copy(rows0.at[pl.ds(0, rem)],
                        acc_sh.at[pl.ds(sid * RPT + (RPT // CHUNK) * CHUNK, rem)])
    plsc.subcore_barrier()

    def gather(j, buf):
        pltpu.async_copy(x_hbm.at[src_v.at[j]], rows[buf], gsems[buf])

    def wait_gather(j, buf):
        pltpu.make_async_copy(x_hbm.at[src_v.at[j]], rows[buf], gsems[buf]).wait()

    def scale(j, buf):
        rbuf = rows[buf]

        def body(g, c):
            av16 = a_v[j, pl.ds(g * LANES, LANES)]
            for i in range(LANES):
                avb = jnp.full((LANES,), av16[i], dtype=jnp.float32)
                e = g * LANES + i
                for k in range(FV):
                    sl = (e, pl.ds(k * LANES, LANES))
                    rbuf[sl] = rbuf[sl] * avb
            return c

        lax.fori_loop(0, CHUNK // LANES, body, 0)

    def scatter(j, buf):
        pltpu.sync_copy(rows[buf], acc_sh.at[dst_v.at[j]], add=True)

    def step(j, buf, do_tail):
        wait_gather(j, buf)
        scale(j, buf)
        scatter(j, buf)
        if do_tail:
            gather(j + 2, buf)

    def pair(m, carry):
        j = 2 * m
        step(j, 0, True)
        step(j + 1, 1, True)
        return carry

    for h in range(NCHUNK // HALF):
        if h:
            stage(h)
        gather(0, 0)
        gather(1, 1)
        lax.fori_loop(0, (HALF - 2) // 2, pair, 0)
        step(HALF - 2, 0, False)
        step(HALF - 1, 1, False)
    plsc.subcore_barrier()

    W_ROWS = 624
    pltpu.sync_copy(acc_sh.at[pl.ds(sid * W_ROWS, W_ROWS)],
                    out_hbm.at[cid, pl.ds(sid * W_ROWS, W_ROWS)])

    @pl.when(sid == NS - 1)
    def _tail():
        pltpu.sync_copy(acc_sh.at[pl.ds(NS * W_ROWS, N_NODES - NS * W_ROWS)],
                        out_hbm.at[cid, pl.ds(NS * W_ROWS, N_NODES - NS * W_ROWS)])


@jax.jit
def _spmm_sc(x, src3, dst3, a3):
    mesh = plsc.VectorSubcoreMesh(core_axis_name="c", subcore_axis_name="s")
    return pl.kernel(
        _spmm_body,
        out_type=jax.ShapeDtypeStruct((NC, N_NODES, F), jnp.float32),
        mesh=mesh,
        scratch_types=[
            pltpu.VMEM((HALF, CHUNK), jnp.int32),
            pltpu.VMEM((HALF, CHUNK), jnp.int32),
            pltpu.VMEM((HALF, CHUNK), jnp.float32),
            pltpu.VMEM((CHUNK, F), jnp.float32),
            pltpu.VMEM((CHUNK, F), jnp.float32),
            pltpu.VMEM_SHARED((N_NODES, F), jnp.float32),
            [pltpu.SemaphoreType.DMA] * NBUF,
        ],
    )(x, src3, dst3, a3)


def _finish_body(p_ref, w_ref, b_ref, o_ref):
    acc = p_ref[0] + p_ref[1]
    y = jnp.dot(acc, w_ref[...], preferred_element_type=jnp.float32)
    y = y + b_ref[...]
    o_ref[...] = jnp.where(y >= 0, y, 0.01 * y)


@jax.jit
def _finish_tc(partials, W, b):
    return pl.pallas_call(
        _finish_body,
        out_shape=jax.ShapeDtypeStruct((N_NODES, F), jnp.float32),
    )(partials, W, b.reshape(1, F))


def kernel(node_features, edge_index, adj_values, W, b):
    dst = edge_index[0].astype(jnp.int32)
    src = edge_index[1].astype(jnp.int32)
    pad = E_PAD - N_EDGES
    dst3 = jnp.pad(dst, (0, pad)).reshape(NW, NCHUNK, CHUNK)
    src3 = jnp.pad(src, (0, pad)).reshape(NW, NCHUNK, CHUNK)
    a3 = jnp.pad(adj_values, (0, pad)).reshape(NW, NCHUNK, CHUNK)
    partials = _spmm_sc(node_features, src3, dst3, a3)
    return _finish_tc(partials, W, b)


# speedup vs baseline: 4.4603x; 1.0433x over previous
import jax
import jax.numpy as jnp
from jax import lax
from jax.experimental import pallas as pl
from jax.experimental.pallas import tpu as pltpu
from jax.experimental.pallas import tpu_sc as plsc

N_NODES = 10000
N_EDGES = 320000
F = 128

NC = 2
NS = 16
NW = NC * NS
CHUNK = 64
LANES = 16
FV = F // LANES
NBUF = 2

UNIT = 32
U0, U1 = 7, 3
E_PAD = NS * (U0 + U1) * UNIT * CHUNK
EROWS = E_PAD // CHUNK


def _spmm_body(x_hbm, src_hbm, dst_hbm, a_hbm, out_hbm,
               src_v, dst_v, a_v, rows0, rows1,
               acc_sh, gsems):
    cid = lax.axis_index("c")
    sid = lax.axis_index("s")
    rows = (rows0, rows1)

    n_units = jnp.where(cid == 0, U0, U1)
    row_base = jnp.where(cid == 0, sid * U0, NS * U0 + sid * U1) * UNIT

    def stage(rb):
        pltpu.sync_copy(src_hbm.at[pl.ds(rb, UNIT)], src_v)
        pltpu.sync_copy(dst_hbm.at[pl.ds(rb, UNIT)], dst_v)
        pltpu.sync_copy(a_hbm.at[pl.ds(rb, UNIT)], a_v)

    zero = jnp.zeros((LANES,), jnp.float32)

    def zrow(i, carry):
        for k in range(FV):
            rows0[i, pl.ds(k * LANES, LANES)] = zero
        return carry

    lax.fori_loop(0, CHUNK, zrow, 0)
    RPT = N_NODES // NS
    for t in range(RPT // CHUNK):
        pltpu.sync_copy(rows0, acc_sh.at[pl.ds(sid * RPT + t * CHUNK, CHUNK)])
    rem = RPT % CHUNK
    if rem:
        pltpu.sync_copy(rows0.at[pl.ds(0, rem)],
                        acc_sh.at[pl.ds(sid * RPT + (RPT // CHUNK) * CHUNK, rem)])
    plsc.subcore_barrier()

    def gather(j, buf):
        pltpu.async_copy(x_hbm.at[src_v.at[j]], rows[buf], gsems[buf])

    def wait_gather(j, buf):
        pltpu.make_async_copy(x_hbm.at[src_v.at[j]], rows[buf], gsems[buf]).wait()

    def scale(j, buf):
        rbuf = rows[buf]

        def body(g, c):
            av16 = a_v[j, pl.ds(g * LANES, LANES)]
            for i in range(LANES):
                avb = jnp.full((LANES,), av16[i], dtype=jnp.float32)
                e = g * LANES + i
                for k in range(FV):
                    sl = (e, pl.ds(k * LANES, LANES))
                    rbuf[sl] = rbuf[sl] * avb
            return c

        lax.fori_loop(0, CHUNK // LANES, body, 0)

    def scatter(j, buf):
        pltpu.sync_copy(rows[buf], acc_sh.at[dst_v.at[j]], add=True)

    def step(j, buf, do_tail):
        wait_gather(j, buf)
        scale(j, buf)
        scatter(j, buf)
        if do_tail:
            gather(j + 2, buf)

    def pair(m, carry):
        j = 2 * m
        step(j, 0, True)
        step(j + 1, 1, True)
        return carry

    def unit_body(u, carry):
        stage(row_base + u * UNIT)
        gather(0, 0)
        gather(1, 1)
        lax.fori_loop(0, (UNIT - 2) // 2, pair, 0)
        step(UNIT - 2, 0, False)
        step(UNIT - 1, 1, False)
        return carry

    lax.fori_loop(0, n_units, unit_body, 0)
    plsc.subcore_barrier()

    W_ROWS = 624
    pltpu.sync_copy(acc_sh.at[pl.ds(sid * W_ROWS, W_ROWS)],
                    out_hbm.at[cid, pl.ds(sid * W_ROWS, W_ROWS)])

    @pl.when(sid == NS - 1)
    def _tail():
        pltpu.sync_copy(acc_sh.at[pl.ds(NS * W_ROWS, N_NODES - NS * W_ROWS)],
                        out_hbm.at[cid, pl.ds(NS * W_ROWS, N_NODES - NS * W_ROWS)])


@jax.jit
def _spmm_sc(x, src3, dst3, a3):
    mesh = plsc.VectorSubcoreMesh(core_axis_name="c", subcore_axis_name="s")
    return pl.kernel(
        _spmm_body,
        out_type=jax.ShapeDtypeStruct((NC, N_NODES, F), jnp.float32),
        mesh=mesh,
        scratch_types=[
            pltpu.VMEM((UNIT, CHUNK), jnp.int32),
            pltpu.VMEM((UNIT, CHUNK), jnp.int32),
            pltpu.VMEM((UNIT, CHUNK), jnp.float32),
            pltpu.VMEM((CHUNK, F), jnp.float32),
            pltpu.VMEM((CHUNK, F), jnp.float32),
            pltpu.VMEM_SHARED((N_NODES, F), jnp.float32),
            [pltpu.SemaphoreType.DMA] * NBUF,
        ],
    )(x, src3, dst3, a3)


def _finish_body(p_ref, w_ref, b_ref, o_ref):
    acc = p_ref[0] + p_ref[1]
    y = jnp.dot(acc, w_ref[...], preferred_element_type=jnp.float32)
    y = y + b_ref[...]
    o_ref[...] = jnp.where(y >= 0, y, 0.01 * y)


@jax.jit
def _finish_tc(partials, W, b):
    return pl.pallas_call(
        _finish_body,
        out_shape=jax.ShapeDtypeStruct((N_NODES, F), jnp.float32),
    )(partials, W, b.reshape(1, F))


def kernel(node_features, edge_index, adj_values, W, b):
    dst = edge_index[0].astype(jnp.int32)
    src = edge_index[1].astype(jnp.int32)
    pad = E_PAD - N_EDGES
    dst3 = jnp.pad(dst, (0, pad)).reshape(EROWS, CHUNK)
    src3 = jnp.pad(src, (0, pad)).reshape(EROWS, CHUNK)
    a3 = jnp.pad(adj_values, (0, pad)).reshape(EROWS, CHUNK)
    partials = _spmm_sc(node_features, src3, dst3, a3)
    return _finish_tc(partials, W, b)
